# baseline (speedup 1.0000x reference)
import functools

import jax
import jax.numpy as jnp
from jax import lax
from jax.experimental import pallas as pl
from jax.experimental.pallas import tpu as pltpu
from jax.experimental.pallas import tpu_sc as plsc

N = 10000
E = 320000
Q = 128
DT = 0.1
MAX_XI = 75.0

NC = 2
NS = 16
NW = NC * NS
QPW = Q // NW
NPW = QPW * N
KE = 2560
NCH = E // KE
UNROLL = 2
IPV = KE // (16 * UNROLL)
KN = 2000
IPV3 = KN // 16


@functools.partial(
    pl.kernel,
    mesh=plsc.VectorSubcoreMesh(core_axis_name="c", subcore_axis_name="s"),
    out_type=jax.ShapeDtypeStruct((Q * N,), jnp.float32),
    compiler_params=pltpu.CompilerParams(needs_layout_passes=False),
    scratch_types=[
        pltpu.VMEM((NPW,), jnp.float32),
        pltpu.VMEM((NPW,), jnp.float32),
        pltpu.VMEM((N + 16,), jnp.float32),
        pltpu.VMEM((N + 16,), jnp.float32),
        pltpu.VMEM((2 * KE,), jnp.int32),
        pltpu.VMEM((2 * KE,), jnp.int32),
        pltpu.VMEM((2 * KE,), jnp.float32),
        pltpu.VMEM((KN,), jnp.float32),
        pltpu.VMEM((KN,), jnp.float32),
        pltpu.VMEM((KN,), jnp.float32),
        pltpu.SemaphoreType.DMA,
        pltpu.SemaphoreType.DMA,
        pltpu.SemaphoreType.DMA,
        pltpu.SemaphoreType.DMA,
        pltpu.SemaphoreType.DMA,
        pltpu.SemaphoreType.DMA,
    ],
)
def _sc_step(fP, collP, srcP, w_hbm, src_hbm, dst_hbm, out,
             f_v, acc_v, deg_v, s_v, esrc, edst, ew, cb, sb, ob,
             sem_s0, sem_s1, sem_d0, sem_d1, sem_w0, sem_w1):
    wid = lax.axis_index("s") * NC + lax.axis_index("c")
    base = wid * NPW
    sem_s = (sem_s0, sem_s1)
    sem_d = (sem_d0, sem_d1)
    sem_w = (sem_w0, sem_w1)

    zeros = jnp.zeros((16,), jnp.float32)
    ones = jnp.full((16,), 1.0, jnp.float32)

    pltpu.sync_copy(fP.at[pl.ds(base, NPW)], f_v)

    def _init_f(i, c):
        sl = pl.ds(i * 16, 16)
        f_v[sl] = jnp.maximum(f_v[sl], 0.0)
        acc_v[sl] = zeros
        return c

    lax.fori_loop(0, NPW // 16, _init_f, 0)

    def _init_n(i, c):
        sl = pl.ds(i * 16, 16)
        deg_v[sl] = zeros
        s_v[sl] = zeros
        return c

    lax.fori_loop(0, (N + 16) // 16, _init_n, 0)

    def _p1_start(j, slot):
        pltpu.make_async_copy(src_hbm.at[pl.ds(j * KE, KE)],
                              esrc.at[pl.ds(slot * KE, KE)], sem_s[slot]).start()

    def _p1_wait(j, slot):
        pltpu.make_async_copy(src_hbm.at[pl.ds(j * KE, KE)],
                              esrc.at[pl.ds(slot * KE, KE)], sem_s[slot]).wait()

    def _p1_proc(slot):
        def ib(i, c):
            b = slot * KE + i * 64
            for u in range(4):
                idx = esrc[pl.ds(b + u * 16, 16)]
                plsc.addupdate_scatter(deg_v, [idx], ones)
            return c

        lax.fori_loop(0, KE // 64, ib, 0)

    def _pipeline(start, wait, process):
        start(0, 0)

        def body(g, carry):
            j0 = g * 2

            @pl.when(j0 + 1 < NCH)
            def _():
                start(j0 + 1, 1)

            wait(j0, 0)
            process(0)

            @pl.when(j0 + 2 < NCH)
            def _():
                start(j0 + 2, 0)

            @pl.when(j0 + 1 < NCH)
            def _():
                wait(j0 + 1, 1)
                process(1)

            return carry

        lax.fori_loop(0, (NCH + 1) // 2, body, 0)

    _pipeline(_p1_start, _p1_wait, _p1_proc)

    def _inv(i, c):
        sl = pl.ds(i * 16, 16)
        deg_v[sl] = 1.0 / deg_v[sl]
        return c

    lax.fori_loop(0, N // 16, _inv, 0)

    def _p2_start(j, slot):
        sl_h = pl.ds(j * KE, KE)
        sl_v = pl.ds(slot * KE, KE)
        pltpu.make_async_copy(src_hbm.at[sl_h], esrc.at[sl_v], sem_s[slot]).start()
        pltpu.make_async_copy(dst_hbm.at[sl_h], edst.at[sl_v], sem_d[slot]).start()
        pltpu.make_async_copy(w_hbm.at[sl_h], ew.at[sl_v], sem_w[slot]).start()

    def _p2_wait(j, slot):
        sl_h = pl.ds(j * KE, KE)
        sl_v = pl.ds(slot * KE, KE)
        pltpu.make_async_copy(src_hbm.at[sl_h], esrc.at[sl_v], sem_s[slot]).wait()
        pltpu.make_async_copy(dst_hbm.at[sl_h], edst.at[sl_v], sem_d[slot]).wait()
        pltpu.make_async_copy(w_hbm.at[sl_h], ew.at[sl_v], sem_w[slot]).wait()

    def _p2_proc(slot):
        def ib(i, carry):
            b = slot * KE + i * (16 * UNROLL)
            st = []
            for u in range(UNROLL):
                sl = pl.ds(b + u * 16, 16)
                s = esrc[sl]
                d = edst[sl]
                wv = ew[sl]
                c = wv * plsc.load_gather(deg_v, [s])
                s4 = s * 4
                d4 = d * 4
                fd = [plsc.load_gather(f_v, [d4 + q]) for q in range(QPW)]
                fs = [plsc.load_gather(f_v, [s4 + q]) for q in range(QPW)]
                st.append((s, d, s4, d4, c, fd, fs))
            for s, d, s4, d4, c, fd, fs in st:
                plsc.addupdate_scatter(s_v, [s], c)
                plsc.addupdate_scatter(s_v, [d], c)
                for q in range(QPW):
                    plsc.addupdate_scatter(acc_v, [s4 + q], c * fd[q])
                    plsc.addupdate_scatter(acc_v, [d4 + q], c * fs[q])
            return carry

        lax.fori_loop(0, IPV, ib, 0)

    _pipeline(_p2_start, _p2_wait, _p2_proc)

    lane = lax.iota(jnp.int32, 16)
    xi_vec = (wid * QPW + lane % 4).astype(jnp.float32) * jnp.float32(
        MAX_XI / (Q - 1))
    sperm = lane >> 2

    def ck_body(ck, carry):
        hb = base + ck * KN
        pltpu.sync_copy(collP.at[pl.ds(hb, KN)], cb)
        pltpu.sync_copy(srcP.at[pl.ds(hb, KN)], sb)

        def ib(i, cc):
            sl = pl.ds(i * 16, 16)
            lo = pl.ds(ck * KN + i * 16, 16)
            fl = f_v[lo]
            sv16 = s_v[pl.ds(ck * (KN // 4) + i * 4, 16)]
            sexp = lax.gather(
                sv16, sperm.reshape(16, 1),
                lax.GatherDimensionNumbers(offset_dims=(),
                                           collapsed_slice_dims=(0,),
                                           start_index_map=(0,)),
                (1,), mode=lax.GatherScatterMode.PROMISE_IN_BOUNDS)
            tr = xi_vec * (acc_v[lo] - sexp * fl)
            o = fl - DT * (tr - cb[sl] - sb[sl])
            ob[sl] = jnp.maximum(o, 0.0)
            return cc

        lax.fori_loop(0, IPV3, ib, 0)
        pltpu.sync_copy(ob, out.at[pl.ds(hb, KN)])
        return carry

    lax.fori_loop(0, NPW // KN, ck_body, 0)


def kernel(f_distribution, collision_term, source_term, edge_weight, edge_index):
    def pack(x):
        return jnp.transpose(x.reshape(N, NW, QPW), (1, 0, 2)).reshape(-1)

    fP = pack(f_distribution)
    collP = pack(collision_term)
    srcP = pack(source_term)
    src = edge_index[0].astype(jnp.int32)
    dst = edge_index[1].astype(jnp.int32)
    outP = _sc_step(fP, collP, srcP, edge_weight, src, dst)
    return jnp.transpose(outP.reshape(NW, N, QPW), (1, 0, 2)).reshape(N, Q)
